# baseline (speedup 1.0000x reference)
import jax
import jax.numpy as jnp
from jax.experimental import pallas as pl
from jax.experimental.pallas import tpu as pltpu

B = 64
N_FRAMES = 128
FRAME_LEN = 256
T = N_FRAMES * FRAME_LEN
N_RATES = 16
FPB = 8
COLS = FPB * FRAME_LEN
GRID = N_FRAMES // FPB


def _body(x_ref, mask_ref, rd_ref, ic_ref, w_ref, b_ref, s_ref, out_ref,
          u_ref):
    i = pl.program_id(0)

    @pl.when(i == 0)
    def _init():
        u_ref[...] = jnp.zeros_like(u_ref)

    xb = x_ref[...]
    eb = jnp.dot(xb * xb, s_ref[...],
                 preferred_element_type=jnp.float32)
    em = eb * mask_ref[0] * (1.0 / FRAME_LEN)
    u_ref[...] += jnp.dot(em, w_ref[...],
                          preferred_element_type=jnp.float32)

    @pl.when(i == GRID - 1)
    def _fin():
        rd = rd_ref[...]
        m = jnp.max(rd, axis=-1, keepdims=True)
        lane = jax.lax.broadcasted_iota(jnp.int32, rd.shape, 1)
        idx = jnp.min(jnp.where(rd == m, lane, N_RATES), axis=-1, keepdims=True)
        rate = 0.5 + 0.1 * idx.astype(jnp.float32)

        logits = rate * rate * u_ref[...] + b_ref[...]
        lmax = jnp.max(logits, axis=-1, keepdims=True)
        ex = jnp.exp(logits - lmax)
        sal = ex / jnp.sum(ex, axis=-1, keepdims=True)

        onehot = (lane == ic_ref[...]).astype(jnp.float32)
        sal_ic = jnp.sum(sal * onehot, axis=-1)
        l1 = 1.0 - sal_ic
        mult = m[:, 0] * jnp.log(m[:, 0])
        loss1 = jnp.sum(l1 * mult) * (1.0 / B)

        ent = jnp.sum(-rd * jnp.log(rd + 1e-12)) * (1.0 / B)
        out_ref[...] = jnp.reshape(loss1 - 0.01 * ent, (1, 1))


def kernel(x, rate_distribution, mask_sample, intent_cats, W_sal, b_sal):
    mask3 = mask_sample.reshape(B, GRID, FPB).transpose(1, 0, 2)
    ic = intent_cats.astype(jnp.int32).reshape(B, 1)
    b2 = b_sal.reshape(1, N_RATES)
    s = (jax.lax.broadcasted_iota(jnp.int32, (COLS, FPB), 0) // FRAME_LEN
         == jax.lax.broadcasted_iota(jnp.int32, (COLS, FPB), 1)
         ).astype(jnp.float32)

    out = pl.pallas_call(
        _body,
        grid=(GRID,),
        in_specs=[
            pl.BlockSpec((B, COLS), lambda i: (0, i)),
            pl.BlockSpec((1, B, FPB), lambda i: (i, 0, 0)),
            pl.BlockSpec((B, N_RATES), lambda i: (0, 0)),
            pl.BlockSpec((B, 1), lambda i: (0, 0)),
            pl.BlockSpec((FPB, N_RATES), lambda i: (i, 0)),
            pl.BlockSpec((1, N_RATES), lambda i: (0, 0)),
            pl.BlockSpec((COLS, FPB), lambda i: (0, 0)),
        ],
        out_specs=pl.BlockSpec((1, 1), lambda i: (0, 0)),
        out_shape=jax.ShapeDtypeStruct((1, 1), jnp.float32),
        scratch_shapes=[pltpu.VMEM((B, N_RATES), jnp.float32)],
    )(x, mask3, rate_distribution, ic, W_sal, b2, s)
    return out[0, 0]


# speedup vs baseline: 1.5076x; 1.5076x over previous
import jax
import jax.numpy as jnp
from jax.experimental import pallas as pl
from jax.experimental.pallas import tpu as pltpu

B = 64
N_FRAMES = 128
FRAME_LEN = 256
T = N_FRAMES * FRAME_LEN
N_RATES = 16
FPB = 16
COLS = FPB * FRAME_LEN
NCHUNK = N_FRAMES // FPB
NBUF = 4


def _copy(x_ref, bufs, sems, c):
    j = c % NBUF
    return pltpu.make_async_copy(
        x_ref.at[:, pl.ds(c * COLS, COLS)], bufs.at[j], sems.at[j])


def _body(x_ref, mask_ref, rd_ref, ic_ref, w_ref, b_ref, s_ref, out_ref,
          bufs, sems):
    for c in range(NBUF):
        _copy(x_ref, bufs, sems, c).start()

    u = jnp.zeros((B, N_RATES), jnp.float32)
    for c in range(NCHUNK):
        _copy(x_ref, bufs, sems, c).wait()
        xb = bufs[c % NBUF]
        eb = jnp.dot(xb * xb, s_ref[...],
                     preferred_element_type=jnp.float32)
        em = eb * mask_ref[c] * (1.0 / FRAME_LEN)
        u = u + jnp.dot(em, w_ref[pl.ds(c * FPB, FPB), :],
                        preferred_element_type=jnp.float32)
        if c + NBUF < NCHUNK:
            _copy(x_ref, bufs, sems, c + NBUF).start()

    rd = rd_ref[...]
    m = jnp.max(rd, axis=-1, keepdims=True)
    lane = jax.lax.broadcasted_iota(jnp.int32, rd.shape, 1)
    idx = jnp.min(jnp.where(rd == m, lane, N_RATES), axis=-1, keepdims=True)
    rate = 0.5 + 0.1 * idx.astype(jnp.float32)

    logits = rate * rate * u + b_ref[...]
    lmax = jnp.max(logits, axis=-1, keepdims=True)
    ex = jnp.exp(logits - lmax)
    sal = ex / jnp.sum(ex, axis=-1, keepdims=True)

    onehot = (lane == ic_ref[...]).astype(jnp.float32)
    sal_ic = jnp.sum(sal * onehot, axis=-1)
    l1 = 1.0 - sal_ic
    mult = m[:, 0] * jnp.log(m[:, 0])
    loss1 = jnp.sum(l1 * mult) * (1.0 / B)

    ent = jnp.sum(-rd * jnp.log(rd + 1e-12)) * (1.0 / B)
    out_ref[...] = jnp.reshape(loss1 - 0.01 * ent, (1, 1))


def kernel(x, rate_distribution, mask_sample, intent_cats, W_sal, b_sal):
    mask3 = mask_sample.reshape(B, NCHUNK, FPB).transpose(1, 0, 2)
    ic = intent_cats.astype(jnp.int32).reshape(B, 1)
    b2 = b_sal.reshape(1, N_RATES)
    s = (jax.lax.broadcasted_iota(jnp.int32, (COLS, FPB), 0) // FRAME_LEN
         == jax.lax.broadcasted_iota(jnp.int32, (COLS, FPB), 1)
         ).astype(jnp.float32)

    vm = pltpu.VMEM
    out = pl.pallas_call(
        _body,
        in_specs=[
            pl.BlockSpec(memory_space=pl.ANY),
            pl.BlockSpec(memory_space=vm),
            pl.BlockSpec(memory_space=vm),
            pl.BlockSpec(memory_space=vm),
            pl.BlockSpec(memory_space=vm),
            pl.BlockSpec(memory_space=vm),
            pl.BlockSpec(memory_space=vm),
        ],
        out_specs=pl.BlockSpec(memory_space=vm),
        out_shape=jax.ShapeDtypeStruct((1, 1), jnp.float32),
        scratch_shapes=[pltpu.VMEM((NBUF, B, COLS), jnp.float32),
                        pltpu.SemaphoreType.DMA((NBUF,))],
    )(x, mask3, rate_distribution, ic, W_sal, b2, s)
    return out[0, 0]


# speedup vs baseline: 1.5249x; 1.0114x over previous
import jax
import jax.numpy as jnp
from jax.experimental import pallas as pl
from jax.experimental.pallas import tpu as pltpu

B = 64
N_FRAMES = 128
FRAME_LEN = 256
T = N_FRAMES * FRAME_LEN
N_RATES = 16
FPB = 16
COLS = FPB * FRAME_LEN
NCHUNK = N_FRAMES // FPB
NBUF = 8


def _copy(x_ref, bufs, sems, c):
    j = c % NBUF
    return pltpu.make_async_copy(
        x_ref.at[:, pl.ds(c * COLS, COLS)], bufs.at[j], sems.at[j])


def _body(x_ref, mask_ref, rd_ref, ic_ref, w_ref, b_ref, s_ref, out_ref,
          bufs, sems):
    for c in range(NBUF):
        _copy(x_ref, bufs, sems, c).start()

    u = jnp.zeros((B, N_RATES), jnp.float32)
    for c in range(NCHUNK):
        _copy(x_ref, bufs, sems, c).wait()
        xb = bufs[c % NBUF]
        eb = jnp.dot(xb * xb, s_ref[...],
                     preferred_element_type=jnp.float32)
        em = eb * mask_ref[c] * (1.0 / FRAME_LEN)
        u = u + jnp.dot(em, w_ref[pl.ds(c * FPB, FPB), :],
                        preferred_element_type=jnp.float32)
        if c + NBUF < NCHUNK:
            _copy(x_ref, bufs, sems, c + NBUF).start()

    rd = rd_ref[...]
    m = jnp.max(rd, axis=-1, keepdims=True)
    lane = jax.lax.broadcasted_iota(jnp.int32, rd.shape, 1)
    idx = jnp.min(jnp.where(rd == m, lane, N_RATES), axis=-1, keepdims=True)
    rate = 0.5 + 0.1 * idx.astype(jnp.float32)

    logits = rate * rate * u + b_ref[...]
    lmax = jnp.max(logits, axis=-1, keepdims=True)
    ex = jnp.exp(logits - lmax)
    sal = ex / jnp.sum(ex, axis=-1, keepdims=True)

    onehot = (lane == ic_ref[...]).astype(jnp.float32)
    sal_ic = jnp.sum(sal * onehot, axis=-1)
    l1 = 1.0 - sal_ic
    mult = m[:, 0] * jnp.log(m[:, 0])
    loss1 = jnp.sum(l1 * mult) * (1.0 / B)

    ent = jnp.sum(-rd * jnp.log(rd + 1e-12)) * (1.0 / B)
    out_ref[...] = jnp.reshape(loss1 - 0.01 * ent, (1, 1))


def kernel(x, rate_distribution, mask_sample, intent_cats, W_sal, b_sal):
    mask3 = mask_sample.reshape(B, NCHUNK, FPB).transpose(1, 0, 2)
    ic = intent_cats.astype(jnp.int32).reshape(B, 1)
    b2 = b_sal.reshape(1, N_RATES)
    s = (jax.lax.broadcasted_iota(jnp.int32, (COLS, FPB), 0) // FRAME_LEN
         == jax.lax.broadcasted_iota(jnp.int32, (COLS, FPB), 1)
         ).astype(jnp.float32)

    vm = pltpu.VMEM
    out = pl.pallas_call(
        _body,
        in_specs=[
            pl.BlockSpec(memory_space=pl.ANY),
            pl.BlockSpec(memory_space=vm),
            pl.BlockSpec(memory_space=vm),
            pl.BlockSpec(memory_space=vm),
            pl.BlockSpec(memory_space=vm),
            pl.BlockSpec(memory_space=vm),
            pl.BlockSpec(memory_space=vm),
        ],
        out_specs=pl.BlockSpec(memory_space=vm),
        out_shape=jax.ShapeDtypeStruct((1, 1), jnp.float32),
        scratch_shapes=[pltpu.VMEM((NBUF, B, COLS), jnp.float32),
                        pltpu.SemaphoreType.DMA((NBUF,))],
    )(x, mask3, rate_distribution, ic, W_sal, b2, s)
    return out[0, 0]
